# baseline (speedup 1.0000x reference)
import functools

import jax
import jax.numpy as jnp
from jax import lax
from jax.experimental import pallas as pl
from jax.experimental.pallas import tpu as pltpu
from jax.experimental.pallas import tpu_sc as plsc

B = 8
K = 4096
C = 256
N = 512

_NC, _NS = 2, 16
_NW = _NC * _NS
_ROWS = B * C
_RPW = _ROWS // _NW
_TPB = _NW // B


def _fps_body(x_ref, y_ref, z_ref, inds_ref, nx_ref, ny_ref, nz_ref):
    x = x_ref[...]
    y = y_ref[...]
    z = z_ref[...]
    iota = lax.broadcasted_iota(jnp.int32, (B, K), 1)

    lx = x[:, 0:1]
    ly = y[:, 0:1]
    lz = z[:, 0:1]
    inds_ref[0:1, :] = jnp.zeros((1, B), jnp.int32)
    nx_ref[0:1, :] = lx.T
    ny_ref[0:1, :] = ly.T
    nz_ref[0:1, :] = lz.T

    dists0 = jnp.full((B, K), 1e10, dtype=jnp.float32)

    def body(i, carry):
        dists, lx, ly, lz = carry
        dx = x - lx
        dy = y - ly
        dz = z - lz
        d = dx * dx + dy * dy + dz * dz
        dists = jnp.minimum(dists, d)
        m = jnp.max(dists, axis=1, keepdims=True)
        idx = jnp.min(jnp.where(dists == m, iota, K), axis=1, keepdims=True)
        sel = iota == idx
        lx = jnp.sum(jnp.where(sel, x, 0.0), axis=1, keepdims=True)
        ly = jnp.sum(jnp.where(sel, y, 0.0), axis=1, keepdims=True)
        lz = jnp.sum(jnp.where(sel, z, 0.0), axis=1, keepdims=True)
        inds_ref[pl.ds(i, 1), :] = idx.T
        nx_ref[pl.ds(i, 1), :] = lx.T
        ny_ref[pl.ds(i, 1), :] = ly.T
        nz_ref[pl.ds(i, 1), :] = lz.T
        return dists, lx, ly, lz

    lax.fori_loop(1, N, body, (dists0, lx, ly, lz))


@jax.jit
def _fps(x, y, z):
    out_shapes = (
        jax.ShapeDtypeStruct((N, B), jnp.int32),
        jax.ShapeDtypeStruct((N, B), jnp.float32),
        jax.ShapeDtypeStruct((N, B), jnp.float32),
        jax.ShapeDtypeStruct((N, B), jnp.float32),
    )
    return pl.pallas_call(
        _fps_body,
        out_shape=out_shapes,
    )(x, y, z)


def _gather_body(feat_hbm, inds_hbm, out_hbm,
                 inds_v, idx0, idx1, idx2, idx3, row_v, sem):
    wid = lax.axis_index("s") * _NC + lax.axis_index("c")
    b = wid // _TPB
    c0 = (wid % _TPB) * _RPW
    pltpu.sync_copy(inds_hbm.at[pl.ds(b * N, N)], inds_v)
    base0 = b * (C * K) + c0 * K

    idx_bufs = (idx0, idx1, idx2, idx3)

    def cbody(cc, carry):
        base = base0 + cc * K
        for j, idx_v in enumerate(idx_bufs):
            for s in range(8):
                idx_v[pl.ds(s * 16, 16)] = inds_v[pl.ds(j * 128 + s * 16, 16)] + base
        for j, idx_v in enumerate(idx_bufs):
            pltpu.async_copy(feat_hbm.at[idx_v],
                             row_v.at[pl.ds(j * 128, 128)], sem).wait()
        pltpu.sync_copy(row_v, out_hbm.at[pl.ds((b * C + c0 + cc) * N, N)])
        return carry

    lax.fori_loop(0, _RPW, cbody, 0)


_feat_gather = pl.kernel(
    _gather_body,
    out_type=jax.ShapeDtypeStruct((B * C * N,), jnp.float32),
    mesh=plsc.VectorSubcoreMesh(core_axis_name="c", subcore_axis_name="s"),
    scratch_types=[
        pltpu.VMEM((N,), jnp.int32),
        pltpu.VMEM((128,), jnp.int32),
        pltpu.VMEM((128,), jnp.int32),
        pltpu.VMEM((128,), jnp.int32),
        pltpu.VMEM((128,), jnp.int32),
        pltpu.VMEM((N,), jnp.float32),
        pltpu.SemaphoreType.DMA,
    ],
)


@jax.jit
def kernel(xyz, features):
    x = xyz[:, :, 0]
    y = xyz[:, :, 1]
    z = xyz[:, :, 2]
    inds_t, nx, ny, nz = _fps(x, y, z)
    sample_inds = inds_t.T
    new_xyz = jnp.stack([nx.T, ny.T, nz.T], axis=-1)
    out_flat = _feat_gather(features.reshape(-1), sample_inds.reshape(-1))
    new_features = out_flat.reshape(B, C, N)
    return new_xyz, new_features, sample_inds


# speedup vs baseline: 1.4281x; 1.4281x over previous
import functools

import jax
import jax.numpy as jnp
from jax import lax
from jax.experimental import pallas as pl
from jax.experimental.pallas import tpu as pltpu
from jax.experimental.pallas import tpu_sc as plsc

B = 8
K = 4096
C = 256
N = 512

_NC, _NS = 2, 16
_NW = _NC * _NS
_ROWS = B * C
_RPW = _ROWS // _NW
_TPB = _NW // B


def _fps_body(x_ref, y_ref, z_ref, inds_ref, nx_ref, ny_ref, nz_ref):
    x = x_ref[...]
    y = y_ref[...]
    z = z_ref[...]
    iota = lax.broadcasted_iota(jnp.int32, (B, K), 1)

    lx = x[:, 0:1]
    ly = y[:, 0:1]
    lz = z[:, 0:1]
    inds_ref[0:1, :] = jnp.zeros((1, B), jnp.int32)
    nx_ref[0:1, :] = lx.T
    ny_ref[0:1, :] = ly.T
    nz_ref[0:1, :] = lz.T

    dists0 = jnp.full((B, K), 1e10, dtype=jnp.float32)

    def body(i, carry):
        dists, lx, ly, lz = carry
        dx = x - lx
        dy = y - ly
        dz = z - lz
        d = dx * dx + dy * dy + dz * dz
        dists = jnp.minimum(dists, d)
        m = jnp.max(dists, axis=1, keepdims=True)
        idx = jnp.min(jnp.where(dists == m, iota, K), axis=1, keepdims=True)
        sel = iota == idx
        lx = jnp.sum(jnp.where(sel, x, 0.0), axis=1, keepdims=True)
        ly = jnp.sum(jnp.where(sel, y, 0.0), axis=1, keepdims=True)
        lz = jnp.sum(jnp.where(sel, z, 0.0), axis=1, keepdims=True)
        inds_ref[pl.ds(i, 1), :] = idx.T
        nx_ref[pl.ds(i, 1), :] = lx.T
        ny_ref[pl.ds(i, 1), :] = ly.T
        nz_ref[pl.ds(i, 1), :] = lz.T
        return dists, lx, ly, lz

    lax.fori_loop(1, N, body, (dists0, lx, ly, lz))


@jax.jit
def _fps(x, y, z):
    out_shapes = (
        jax.ShapeDtypeStruct((N, B), jnp.int32),
        jax.ShapeDtypeStruct((N, B), jnp.float32),
        jax.ShapeDtypeStruct((N, B), jnp.float32),
        jax.ShapeDtypeStruct((N, B), jnp.float32),
    )
    return pl.pallas_call(
        _fps_body,
        out_shape=out_shapes,
    )(x, y, z)


_GC = 4
_NG = _RPW // _GC
_CHUNK = 128
_CPG = _GC * N // _CHUNK


def _gather_body(feat_hbm, inds_hbm, out_hbm,
                 inds_v, idxA, idxB, rowA, rowB, semA, semB):
    wid = lax.axis_index("s") * _NC + lax.axis_index("c")
    b = wid // _TPB
    c0 = (wid % _TPB) * _RPW
    pltpu.sync_copy(inds_hbm.at[pl.ds(b * N, N)], inds_v)
    base0 = b * (C * K) + c0 * K

    def build(idxbuf, g):
        for j in range(_CPG):
            base = base0 + (g * _GC + j // 4) * K
            for s in range(8):
                idxbuf[j, pl.ds(s * 16, 16)] = (
                    inds_v[pl.ds((j % 4) * _CHUNK + s * 16, 16)] + base)

    def fire(idxbuf, rowbuf, sem):
        for j in range(_CPG):
            pltpu.async_copy(feat_hbm.at[idxbuf.at[j]],
                             rowbuf.at[pl.ds(j * _CHUNK, _CHUNK)], sem)

    def drain(rowbuf, sem):
        pltpu.make_async_copy(feat_hbm.at[pl.ds(0, _GC * N)], rowbuf, sem).wait()

    def out(rowbuf, g):
        pltpu.sync_copy(rowbuf,
                        out_hbm.at[pl.ds((b * C + c0 + g * _GC) * N, _GC * N)])

    build(idxA, 0)
    fire(idxA, rowA, semA)

    def body(it, carry):
        gA = 2 * it
        build(idxB, gA + 1)
        fire(idxB, rowB, semB)
        drain(rowA, semA)
        out(rowA, gA)

        @pl.when(it < _NG // 2 - 1)
        def _():
            build(idxA, gA + 2)
            fire(idxA, rowA, semA)

        drain(rowB, semB)
        out(rowB, gA + 1)
        return carry

    lax.fori_loop(0, _NG // 2, body, 0)


_feat_gather = pl.kernel(
    _gather_body,
    out_type=jax.ShapeDtypeStruct((B * C * N,), jnp.float32),
    mesh=plsc.VectorSubcoreMesh(core_axis_name="c", subcore_axis_name="s"),
    scratch_types=[
        pltpu.VMEM((N,), jnp.int32),
        pltpu.VMEM((_CPG, _CHUNK), jnp.int32),
        pltpu.VMEM((_CPG, _CHUNK), jnp.int32),
        pltpu.VMEM((_GC * N,), jnp.float32),
        pltpu.VMEM((_GC * N,), jnp.float32),
        pltpu.SemaphoreType.DMA,
        pltpu.SemaphoreType.DMA,
    ],
)


@jax.jit
def kernel(xyz, features):
    x = xyz[:, :, 0]
    y = xyz[:, :, 1]
    z = xyz[:, :, 2]
    inds_t, nx, ny, nz = _fps(x, y, z)
    sample_inds = inds_t.T
    new_xyz = jnp.stack([nx.T, ny.T, nz.T], axis=-1)
    out_flat = _feat_gather(features.reshape(-1), sample_inds.reshape(-1))
    new_features = out_flat.reshape(B, C, N)
    return new_xyz, new_features, sample_inds
